# baseline (device time: 116969 ns/iter reference)
import jax
import jax.numpy as jnp
from jax import lax
from jax.experimental import pallas as pl
from jax.experimental.pallas import tpu as pltpu

N_DEV = 4
N_EXP = 16
E_PER = 4
CAP = 409.0


def kernel(x, router_W, route_idx, expert_W):
    del router_W
    m, d = x.shape
    e_per, _, h = expert_W.shape

    def body(x_ref, route_ref, w_ref, out_ref,
             xb_ref, wb_ref, tot_ref, cw_ref, cc_ref,
             w_send, w_recv, c_send, c_recv):
        my = lax.axis_index("i")
        left = lax.rem(my - 1 + N_DEV, N_DEV)
        right = lax.rem(my + 1, N_DEV)

        bar = pltpu.get_barrier_semaphore()
        for nbr in (left, right):
            pl.semaphore_signal(bar, inc=1, device_id=(nbr,),
                                device_id_type=pl.DeviceIdType.MESH)
        pl.semaphore_wait(bar, 2)

        lane = lax.broadcasted_iota(jnp.int32, (1, 128), 1)
        ohb = (route_ref[...] == lane).astype(jnp.bfloat16)
        ohf = ohb.astype(jnp.float32)
        totals = jnp.sum(ohf, axis=0, keepdims=True)
        tot_ref[...] = jnp.broadcast_to(totals, (8, 128))

        prefix = jnp.zeros((1, 128), jnp.float32)
        rc0 = pltpu.make_async_remote_copy(
            src_ref=tot_ref, dst_ref=cc_ref.at[0],
            send_sem=c_send.at[0], recv_sem=c_recv.at[0],
            device_id=(right,), device_id_type=pl.DeviceIdType.MESH)
        rc0.start()
        wb_ref[...] = w_ref[...].astype(jnp.bfloat16)
        rc0.wait()
        for hh in range(N_DEV - 1):
            if hh > 0:
                rc = pltpu.make_async_remote_copy(
                    src_ref=cc_ref.at[hh - 1], dst_ref=cc_ref.at[hh],
                    send_sem=c_send.at[hh], recv_sem=c_recv.at[hh],
                    device_id=(right,), device_id_type=pl.DeviceIdType.MESH)
                rc.start()
                rc.wait()
            origin = lax.rem(my - hh - 1 + N_DEV, N_DEV)
            row = cc_ref[hh, 0:1, :]
            prefix = prefix + jnp.where(origin < my, row,
                                        jnp.zeros_like(row))

        send_r = pltpu.make_async_remote_copy(
            src_ref=wb_ref, dst_ref=cw_ref.at[0],
            send_sem=w_send.at[0], recv_sem=w_recv.at[0],
            device_id=(right,), device_id_type=pl.DeviceIdType.MESH)
        send_l = pltpu.make_async_remote_copy(
            src_ref=wb_ref, dst_ref=cw_ref.at[1],
            send_sem=w_send.at[1], recv_sem=w_recv.at[1],
            device_id=(left,), device_id_type=pl.DeviceIdType.MESH)
        send_r.start()
        send_l.start()

        xb_ref[...] = x_ref[...].astype(jnp.bfloat16)
        row_i = lax.broadcasted_iota(jnp.int32, (m, m), 0)
        col_i = lax.broadcasted_iota(jnp.int32, (m, m), 1)
        tril = (col_i < row_i).astype(jnp.bfloat16)
        ranks = jnp.dot(tril, ohb,
                        preferred_element_type=jnp.float32)
        rank = jnp.sum(ranks * ohf, axis=1, keepdims=True)
        offset = jnp.sum(ohf * prefix, axis=1, keepdims=True)
        kept = ((rank + offset) < CAP).astype(jnp.bfloat16)

        def accum_block(wref, origin, init):
            parts = []
            for el in range(E_PER):
                e = origin * E_PER + el
                mask = (route_ref[...] == e).astype(jnp.bfloat16) * kept
                parts.append(xb_ref[...] * mask)
            xm = jnp.concatenate(parts, axis=1)
            w_flat = wref[...].reshape(E_PER * d, h)
            y = jnp.dot(xm, w_flat, preferred_element_type=jnp.float32)
            if init:
                out_ref[...] = y
            else:
                out_ref[...] += y

        accum_block(wb_ref, my, init=True)

        send_r.wait_recv()
        fwd_r = pltpu.make_async_remote_copy(
            src_ref=cw_ref.at[0, 0:2], dst_ref=cw_ref.at[2, 0:2],
            send_sem=w_send.at[2], recv_sem=w_recv.at[2],
            device_id=(right,), device_id_type=pl.DeviceIdType.MESH)
        fwd_r.start()
        accum_block(cw_ref.at[0], left, init=False)

        send_l.wait_recv()
        fwd_l = pltpu.make_async_remote_copy(
            src_ref=cw_ref.at[1, 2:4], dst_ref=cw_ref.at[2, 2:4],
            send_sem=w_send.at[3], recv_sem=w_recv.at[3],
            device_id=(left,), device_id_type=pl.DeviceIdType.MESH)
        fwd_l.start()
        accum_block(cw_ref.at[1], right, init=False)

        fwd_r.wait_recv()
        fwd_l.wait_recv()
        far = lax.rem(my + 2, N_DEV)
        accum_block(cw_ref.at[2], far, init=False)

        for r in (send_r, send_l, fwd_r, fwd_l):
            r.wait_send()

    return pl.pallas_call(
        body,
        out_shape=jax.ShapeDtypeStruct((m, h), jnp.float32),
        in_specs=[pl.BlockSpec(memory_space=pltpu.VMEM)] * 3,
        out_specs=pl.BlockSpec(memory_space=pltpu.VMEM),
        scratch_shapes=[
            pltpu.VMEM((m, d), jnp.bfloat16),
            pltpu.VMEM((e_per, d, h), jnp.bfloat16),
            pltpu.VMEM((8, 128), jnp.float32),
            pltpu.VMEM((N_DEV - 1, e_per, d, h), jnp.bfloat16),
            pltpu.VMEM((N_DEV - 1, 8, 128), jnp.float32),
            pltpu.SemaphoreType.DMA((4,)),
            pltpu.SemaphoreType.DMA((4,)),
            pltpu.SemaphoreType.DMA((N_DEV - 1,)),
            pltpu.SemaphoreType.DMA((N_DEV - 1,)),
        ],
        compiler_params=pltpu.CompilerParams(
            collective_id=0, vmem_limit_bytes=100 * 1024 * 1024),
    )(x, route_idx, expert_W)


# device time: 114875 ns/iter; 1.0182x vs baseline; 1.0182x over previous
import jax
import jax.numpy as jnp
from jax import lax
from jax.experimental import pallas as pl
from jax.experimental.pallas import tpu as pltpu

N_DEV = 4
N_EXP = 16
E_PER = 4
CAP = 409.0


def kernel(x, router_W, route_idx, expert_W):
    del router_W
    m, d = x.shape
    e_per, _, h = expert_W.shape

    def body(x_ref, route_ref, w_ref, out_ref,
             xb_ref, wb_ref, tot_ref, cw_ref, cc_ref,
             w_send, w_recv, c_send, c_recv):
        my = lax.axis_index("i")
        left = lax.rem(my - 1 + N_DEV, N_DEV)
        right = lax.rem(my + 1, N_DEV)

        bar = pltpu.get_barrier_semaphore()
        for nbr in (left, right):
            pl.semaphore_signal(bar, inc=1, device_id=(nbr,),
                                device_id_type=pl.DeviceIdType.MESH)
        pl.semaphore_wait(bar, 2)

        lane = lax.broadcasted_iota(jnp.int32, (1, 128), 1)
        ohb = (route_ref[...] == lane).astype(jnp.bfloat16)
        ohf = ohb.astype(jnp.float32)
        totals = jnp.sum(ohf, axis=0, keepdims=True)
        tot_ref[...] = jnp.broadcast_to(totals, (8, 128))

        far = lax.rem(my + 2, N_DEV)
        c_rdmas = []
        for slot, tgt in ((0, right), (1, left), (2, far)):
            rc = pltpu.make_async_remote_copy(
                src_ref=tot_ref, dst_ref=cc_ref.at[slot],
                send_sem=c_send.at[slot], recv_sem=c_recv.at[slot],
                device_id=(tgt,), device_id_type=pl.DeviceIdType.MESH)
            rc.start()
            c_rdmas.append(rc)
        wb_ref[...] = w_ref[...].astype(jnp.bfloat16)
        prefix = jnp.zeros((1, 128), jnp.float32)
        for rc, (slot, origin) in zip(
                c_rdmas, ((0, left), (1, right), (2, far))):
            rc.wait()
            row = cc_ref[slot, 0:1, :]
            prefix = prefix + jnp.where(origin < my, row,
                                        jnp.zeros_like(row))

        send_r = pltpu.make_async_remote_copy(
            src_ref=wb_ref, dst_ref=cw_ref.at[0],
            send_sem=w_send.at[0], recv_sem=w_recv.at[0],
            device_id=(right,), device_id_type=pl.DeviceIdType.MESH)
        send_l = pltpu.make_async_remote_copy(
            src_ref=wb_ref, dst_ref=cw_ref.at[1],
            send_sem=w_send.at[1], recv_sem=w_recv.at[1],
            device_id=(left,), device_id_type=pl.DeviceIdType.MESH)
        send_r.start()
        send_l.start()

        xb_ref[...] = x_ref[...].astype(jnp.bfloat16)
        row_i = lax.broadcasted_iota(jnp.int32, (m, m), 0)
        col_i = lax.broadcasted_iota(jnp.int32, (m, m), 1)
        tril = (col_i < row_i).astype(jnp.bfloat16)
        ranks = jnp.dot(tril, ohb,
                        preferred_element_type=jnp.float32)
        rank = jnp.sum(ranks * ohf, axis=1, keepdims=True)
        offset = jnp.sum(ohf * prefix, axis=1, keepdims=True)
        kept = ((rank + offset) < CAP).astype(jnp.bfloat16)

        def accum_block(wref, origin, init):
            parts = []
            for el in range(E_PER):
                e = origin * E_PER + el
                mask = (route_ref[...] == e).astype(jnp.bfloat16) * kept
                parts.append(xb_ref[...] * mask)
            xm = jnp.concatenate(parts, axis=1)
            w_flat = wref[...].reshape(E_PER * d, h)
            y = jnp.dot(xm, w_flat, preferred_element_type=jnp.float32)
            if init:
                out_ref[...] = y
            else:
                out_ref[...] += y

        accum_block(wb_ref, my, init=True)

        send_r.wait_recv()
        fwd_r = pltpu.make_async_remote_copy(
            src_ref=cw_ref.at[0, 0:2], dst_ref=cw_ref.at[2, 0:2],
            send_sem=w_send.at[2], recv_sem=w_recv.at[2],
            device_id=(right,), device_id_type=pl.DeviceIdType.MESH)
        fwd_r.start()
        accum_block(cw_ref.at[0], left, init=False)

        send_l.wait_recv()
        fwd_l = pltpu.make_async_remote_copy(
            src_ref=cw_ref.at[1, 2:4], dst_ref=cw_ref.at[2, 2:4],
            send_sem=w_send.at[3], recv_sem=w_recv.at[3],
            device_id=(left,), device_id_type=pl.DeviceIdType.MESH)
        fwd_l.start()
        accum_block(cw_ref.at[1], right, init=False)

        fwd_r.wait_recv()
        fwd_l.wait_recv()
        accum_block(cw_ref.at[2], far, init=False)

        for r in (send_r, send_l, fwd_r, fwd_l):
            r.wait_send()

    return pl.pallas_call(
        body,
        out_shape=jax.ShapeDtypeStruct((m, h), jnp.float32),
        in_specs=[pl.BlockSpec(memory_space=pltpu.VMEM)] * 3,
        out_specs=pl.BlockSpec(memory_space=pltpu.VMEM),
        scratch_shapes=[
            pltpu.VMEM((m, d), jnp.bfloat16),
            pltpu.VMEM((e_per, d, h), jnp.bfloat16),
            pltpu.VMEM((8, 128), jnp.float32),
            pltpu.VMEM((N_DEV - 1, e_per, d, h), jnp.bfloat16),
            pltpu.VMEM((N_DEV - 1, 8, 128), jnp.float32),
            pltpu.SemaphoreType.DMA((4,)),
            pltpu.SemaphoreType.DMA((4,)),
            pltpu.SemaphoreType.DMA((N_DEV - 1,)),
            pltpu.SemaphoreType.DMA((N_DEV - 1,)),
        ],
        compiler_params=pltpu.CompilerParams(
            collective_id=0, vmem_limit_bytes=100 * 1024 * 1024),
    )(x, route_idx, expert_W)


# device time: 103665 ns/iter; 1.1283x vs baseline; 1.1081x over previous
import jax
import jax.numpy as jnp
from jax import lax
from jax.experimental import pallas as pl
from jax.experimental.pallas import tpu as pltpu

N_DEV = 4
N_EXP = 16
E_PER = 4
CAP = 409.0


def kernel(x, router_W, route_idx, expert_W):
    del router_W
    m, d = x.shape
    e_per, _, h = expert_W.shape

    def body(x_ref, route_ref, w_ref, out_ref,
             xb_ref, wb_ref, tot_ref, cw_ref, cc_ref,
             w_send, w_recv, c_send, c_recv):
        my = lax.axis_index("i")
        left = lax.rem(my - 1 + N_DEV, N_DEV)
        right = lax.rem(my + 1, N_DEV)

        bar = pltpu.get_barrier_semaphore()
        for nbr in (left, right):
            pl.semaphore_signal(bar, inc=1, device_id=(nbr,),
                                device_id_type=pl.DeviceIdType.MESH)
        pl.semaphore_wait(bar, 2)

        lane = lax.broadcasted_iota(jnp.int32, (1, 128), 1)
        ohb = (route_ref[...] == lane).astype(jnp.bfloat16)
        ohf = ohb.astype(jnp.float32)
        totals = jnp.sum(ohf, axis=0, keepdims=True)
        tot_ref[...] = jnp.broadcast_to(totals, (8, 128))

        far = lax.rem(my + 2, N_DEV)
        c_rdmas = []
        for slot, tgt in ((0, right), (1, left), (2, far)):
            rc = pltpu.make_async_remote_copy(
                src_ref=tot_ref, dst_ref=cc_ref.at[slot],
                send_sem=c_send.at[slot], recv_sem=c_recv.at[slot],
                device_id=(tgt,), device_id_type=pl.DeviceIdType.MESH)
            rc.start()
            c_rdmas.append(rc)
        wb_ref[...] = w_ref[...].astype(jnp.bfloat16)
        prefix = jnp.zeros((1, 128), jnp.float32)
        for rc, (slot, origin) in zip(
                c_rdmas, ((0, left), (1, right), (2, far))):
            rc.wait()
            row = cc_ref[slot, 0:1, :]
            prefix = prefix + jnp.where(origin < my, row,
                                        jnp.zeros_like(row))

        def chunk_send(slot, lo, hi, tgt, sem):
            r = pltpu.make_async_remote_copy(
                src_ref=wb_ref.at[lo:hi], dst_ref=cw_ref.at[slot, lo:hi],
                send_sem=w_send.at[sem], recv_sem=w_recv.at[sem],
                device_id=(tgt,), device_id_type=pl.DeviceIdType.MESH)
            r.start()
            return r

        send_r_a = chunk_send(0, 0, 2, right, 0)
        send_r_b = chunk_send(0, 2, 4, right, 1)
        send_l_a = chunk_send(1, 0, 2, left, 2)
        send_l_b = chunk_send(1, 2, 4, left, 3)

        xb_ref[...] = x_ref[...].astype(jnp.bfloat16)
        row_i = lax.broadcasted_iota(jnp.int32, (m, m), 0)
        col_i = lax.broadcasted_iota(jnp.int32, (m, m), 1)
        tril = (col_i < row_i).astype(jnp.bfloat16)
        ranks = jnp.dot(tril, ohb,
                        preferred_element_type=jnp.float32)
        rank = jnp.sum(ranks * ohf, axis=1, keepdims=True)
        offset = jnp.sum(ohf * prefix, axis=1, keepdims=True)
        kept = ((rank + offset) < CAP).astype(jnp.bfloat16)

        def accum(wref, origin, lo, hi, init=False):
            parts = []
            for el in range(lo, hi):
                e = origin * E_PER + el
                mask = (route_ref[...] == e).astype(jnp.bfloat16) * kept
                parts.append(xb_ref[...] * mask)
            xm = jnp.concatenate(parts, axis=1)
            w_flat = wref[lo:hi].reshape((hi - lo) * d, h)
            y = jnp.dot(xm, w_flat, preferred_element_type=jnp.float32)
            if init:
                out_ref[...] = y
            else:
                out_ref[...] += y

        accum(wb_ref, my, 0, E_PER, init=True)

        send_r_a.wait_recv()
        fwd_r = pltpu.make_async_remote_copy(
            src_ref=cw_ref.at[0, 0:2], dst_ref=cw_ref.at[2, 0:2],
            send_sem=w_send.at[4], recv_sem=w_recv.at[4],
            device_id=(right,), device_id_type=pl.DeviceIdType.MESH)
        fwd_r.start()
        accum(cw_ref.at[0], left, 0, 2)

        send_l_a.wait_recv()
        accum(cw_ref.at[1], right, 0, 2)

        send_l_b.wait_recv()
        fwd_l = pltpu.make_async_remote_copy(
            src_ref=cw_ref.at[1, 2:4], dst_ref=cw_ref.at[2, 2:4],
            send_sem=w_send.at[5], recv_sem=w_recv.at[5],
            device_id=(left,), device_id_type=pl.DeviceIdType.MESH)
        fwd_l.start()
        accum(cw_ref.at[1], right, 2, E_PER)

        send_r_b.wait_recv()
        accum(cw_ref.at[0], left, 2, E_PER)

        fwd_r.wait_recv()
        accum(cw_ref.at[2], far, 0, 2)
        fwd_l.wait_recv()
        accum(cw_ref.at[2], far, 2, E_PER)

        for r in (send_r_a, send_r_b, send_l_a, send_l_b, fwd_r, fwd_l):
            r.wait_send()

    return pl.pallas_call(
        body,
        out_shape=jax.ShapeDtypeStruct((m, h), jnp.float32),
        in_specs=[pl.BlockSpec(memory_space=pltpu.VMEM)] * 3,
        out_specs=pl.BlockSpec(memory_space=pltpu.VMEM),
        scratch_shapes=[
            pltpu.VMEM((m, d), jnp.bfloat16),
            pltpu.VMEM((e_per, d, h), jnp.bfloat16),
            pltpu.VMEM((8, 128), jnp.float32),
            pltpu.VMEM((N_DEV - 1, e_per, d, h), jnp.bfloat16),
            pltpu.VMEM((N_DEV - 1, 8, 128), jnp.float32),
            pltpu.SemaphoreType.DMA((6,)),
            pltpu.SemaphoreType.DMA((6,)),
            pltpu.SemaphoreType.DMA((N_DEV - 1,)),
            pltpu.SemaphoreType.DMA((N_DEV - 1,)),
        ],
        compiler_params=pltpu.CompilerParams(
            collective_id=0, vmem_limit_bytes=100 * 1024 * 1024),
    )(x, route_idx, expert_W)
